# baseline (speedup 1.0000x reference)
import jax
import jax.numpy as jnp
from jax.experimental import pallas as pl
from jax.experimental.pallas import tpu as pltpu

P = 16
S = 4096
D = 512
BLK = 2048
K = S // BLK
NEG_SLOPE = 0.2
EPS = 1e-16
BF = jnp.bfloat16


def _lrelu(v):
    return jnp.where(v >= 0, v, NEG_SLOPE * v)


def _row_dot(vec_row, mat):
    return jax.lax.dot_general(
        vec_row, mat, (((1,), (1,)), ((), ())),
        preferred_element_type=jnp.float32)


def _sample_side(hs, hp, asp_row, maxasp, as_col, ad_col, b):
    e = _lrelu(asp_row + ad_col)
    e_self = _lrelu(as_col + ad_col)
    m = _lrelu(jnp.maximum(as_col, maxasp) + ad_col)
    w = jnp.exp(e - m)
    w_self = jnp.exp(e_self - m)
    inv = 1.0 / (jnp.sum(w, axis=1, keepdims=True) + w_self + EPS)
    out = (jnp.dot(w * inv, hp, preferred_element_type=jnp.float32)
           + (w_self * inv) * hs)
    return jnp.maximum(out + b, 0.0)


def _fused_body(fp_ref, fs_ref, w1_ref, as1_ref, ad1_ref, b1_ref,
                w2_ref, as2_ref, ad2_ref, b2_ref, wfc_ref, bfc_ref,
                h_ref, pred_ref,
                g1_s, gp1_s, wbf_s, hp_s, aspr_s, adp_s, aspc_s,
                m_s, s_s, acc_s):
    l = pl.program_id(0)
    k = pl.program_id(1)

    @pl.when(l == 0)
    def _layer1():
        asrc = as1_ref[...]
        adst = ad1_ref[...]
        b = b1_ref[...]

        @pl.when(k == 0)
        def _():
            wbf = w1_ref[...].astype(BF)
            wbf_s[...] = wbf
            hp0 = jnp.dot(fp_ref[...].astype(BF), wbf,
                          preferred_element_type=jnp.float32)
            hp_s[...] = hp0
            aspr_s[...] = _row_dot(asrc, hp0)
            adp_s[...] = jnp.sum(hp0 * adst, axis=1, keepdims=True)
            aspc_s[...] = jnp.sum(hp0 * asrc, axis=1, keepdims=True)
            m_s[...] = jnp.full_like(m_s, -jnp.inf)
            s_s[...] = jnp.zeros_like(s_s)
            acc_s[...] = jnp.zeros_like(acc_s)

        w = wbf_s[...]
        hp = hp_s[...]
        asp_row = aspr_s[...]
        adp_col = adp_s[...]

        hs = jnp.dot(fs_ref[...].astype(BF), w,
                     preferred_element_type=jnp.float32)
        as_col = jnp.sum(hs * asrc, axis=1, keepdims=True)
        ad_col = jnp.sum(hs * adst, axis=1, keepdims=True)
        as_row = _row_dot(asrc, hs)

        g1_s[pl.ds(k * BLK, BLK), :] = _sample_side(
            hs, hp, asp_row, jnp.max(asp_row), as_col, ad_col, b).astype(BF)

        ep = _lrelu(adp_col + as_row)
        bmax = _lrelu(jnp.max(as_row) + adp_col)
        new_m = jnp.maximum(m_s[...], bmax)
        scale = jnp.exp(m_s[...] - new_m)
        wp = jnp.exp(ep - new_m)
        s_s[...] = s_s[...] * scale + jnp.sum(wp, axis=1, keepdims=True)
        acc_s[...] = (acc_s[...] * scale
                      + jnp.dot(wp, hs, preferred_element_type=jnp.float32))
        m_s[...] = new_m

        @pl.when(k == K - 1)
        def _():
            e_sp = _lrelu(aspc_s[...] + adp_col)
            fm = jnp.maximum(m_s[...], e_sp)
            sc = jnp.exp(m_s[...] - fm)
            wsp = jnp.exp(e_sp - fm)
            den = s_s[...] * sc + wsp + EPS
            accf = acc_s[...] * sc + wsp * hp
            gp1_s[...] = jnp.maximum(accf / den + b, 0.0)

    @pl.when(l == 1)
    def _layer2():
        asrc = as2_ref[...]
        adst = ad2_ref[...]
        b = b2_ref[...]

        @pl.when(k == 0)
        def _():
            wbf = w2_ref[...].astype(BF)
            wbf_s[...] = wbf
            hp0 = jnp.dot(gp1_s[...].astype(BF), wbf,
                          preferred_element_type=jnp.float32)
            hp_s[...] = hp0
            aspr_s[...] = _row_dot(asrc, hp0)

        w = wbf_s[...]
        hp = hp_s[...]
        asp_row = aspr_s[...]

        fs = g1_s[pl.ds(k * BLK, BLK), :]
        hs = jnp.dot(fs, w, preferred_element_type=jnp.float32)
        as_col = jnp.sum(hs * asrc, axis=1, keepdims=True)
        ad_col = jnp.sum(hs * adst, axis=1, keepdims=True)

        g = _sample_side(hs, hp, asp_row, jnp.max(asp_row), as_col,
                         ad_col, b)
        h_ref[...] = g
        pred_ref[...] = (jnp.dot(g.astype(BF), wfc_ref[...],
                                 preferred_element_type=jnp.float32)
                         + bfc_ref[...])


@jax.jit
def _run(x, proxies, W1, a_src1, a_dst1, b1, W2, a_src2, a_dst2, b2,
         W_fc, b_fc):
    C = W_fc.shape[1]
    CP = 128
    wfc = jnp.pad(W_fc, ((0, 0), (0, CP - C))).astype(BF)
    bfc = jnp.pad(b_fc, (0, CP - C))[None, :]

    h2, preds = pl.pallas_call(
        _fused_body,
        grid=(2, K),
        in_specs=[
            pl.BlockSpec((P, D), lambda l, k: (0, 0)),
            pl.BlockSpec((BLK, D), lambda l, k: (k * (1 - l), 0)),
            pl.BlockSpec((D, D), lambda l, k: (0, 0)),
            pl.BlockSpec((1, D), lambda l, k: (0, 0)),
            pl.BlockSpec((1, D), lambda l, k: (0, 0)),
            pl.BlockSpec((1, D), lambda l, k: (0, 0)),
            pl.BlockSpec((D, D), lambda l, k: (0, 0)),
            pl.BlockSpec((1, D), lambda l, k: (0, 0)),
            pl.BlockSpec((1, D), lambda l, k: (0, 0)),
            pl.BlockSpec((1, D), lambda l, k: (0, 0)),
            pl.BlockSpec((D, 128), lambda l, k: (0, 0)),
            pl.BlockSpec((1, 128), lambda l, k: (0, 0)),
        ],
        out_specs=[
            pl.BlockSpec((BLK, D), lambda l, k: (k * l, 0)),
            pl.BlockSpec((BLK, 128), lambda l, k: (k * l, 0)),
        ],
        out_shape=[
            jax.ShapeDtypeStruct((S, D), jnp.float32),
            jax.ShapeDtypeStruct((S, 128), jnp.float32),
        ],
        scratch_shapes=[
            pltpu.VMEM((S, D), BF),
            pltpu.VMEM((P, D), jnp.float32),
            pltpu.VMEM((D, D), BF),
            pltpu.VMEM((P, D), jnp.float32),
            pltpu.VMEM((1, P), jnp.float32),
            pltpu.VMEM((P, 1), jnp.float32),
            pltpu.VMEM((P, 1), jnp.float32),
            pltpu.VMEM((P, 1), jnp.float32),
            pltpu.VMEM((P, 1), jnp.float32),
            pltpu.VMEM((P, D), jnp.float32),
        ],
    )(proxies, x,
      W1, a_src1[None, :], a_dst1[None, :], b1[None, :],
      W2, a_src2[None, :], a_dst2[None, :], b2[None, :], wfc, bfc)

    return preds[:, :C], h2


def kernel(x, proxies, W1, a_src1, a_dst1, b1, W2, a_src2, a_dst2, b2,
           W_fc, b_fc):
    return _run(x, proxies, W1, a_src1, a_dst1, b1,
                W2, a_src2, a_dst2, b2, W_fc, b_fc)


# speedup vs baseline: 1.0414x; 1.0414x over previous
import jax
import jax.numpy as jnp
from jax.experimental import pallas as pl
from jax.experimental.pallas import tpu as pltpu

P = 16
S = 4096
D = 512
BLK = 2048
K = S // BLK
NEG_SLOPE = 0.2
EPS = 1e-16
BF = jnp.bfloat16


def _lrelu(v):
    return jnp.where(v >= 0, v, NEG_SLOPE * v)


def _row_dot(vec_row, mat):
    return jax.lax.dot_general(
        vec_row, mat, (((1,), (1,)), ((), ())),
        preferred_element_type=jnp.float32)


def _sample_side(hs, hp, asp_row, maxasp, as_col, ad_col, b):
    e = _lrelu(asp_row + ad_col)
    e_self = _lrelu(as_col + ad_col)
    m = _lrelu(jnp.maximum(as_col, maxasp) + ad_col)
    w = jnp.exp(e - m)
    w_self = jnp.exp(e_self - m)
    inv = 1.0 / (jnp.sum(w, axis=1, keepdims=True) + w_self + EPS)
    out = (jnp.dot(w * inv, hp, preferred_element_type=jnp.float32)
           + (w_self * inv) * hs)
    return jnp.maximum(out + b, 0.0)


def _fused_body(fp_ref, fs_ref, w1_ref, as1_ref, ad1_ref, b1_ref,
                w2_ref, as2_ref, ad2_ref, b2_ref, wfc_ref, bfc_ref,
                h_ref, pred_ref,
                g1_s, gp1_s, wbf_s, wfcbf_s, hp_s, aspr_s, adp_s, aspc_s,
                m_s, s_s, acc_s):
    l = pl.program_id(0)
    k = pl.program_id(1)

    @pl.when(l == 0)
    def _layer1():
        asrc = as1_ref[...]
        adst = ad1_ref[...]
        b = b1_ref[...]

        @pl.when(k == 0)
        def _():
            wbf = w1_ref[...].astype(BF)
            wbf_s[...] = wbf
            hp0 = jnp.dot(fp_ref[...].astype(BF), wbf,
                          preferred_element_type=jnp.float32)
            hp_s[...] = hp0
            aspr_s[...] = _row_dot(asrc, hp0)
            adp_s[...] = jnp.sum(hp0 * adst, axis=1, keepdims=True)
            aspc_s[...] = jnp.sum(hp0 * asrc, axis=1, keepdims=True)
            m_s[...] = jnp.full_like(m_s, -jnp.inf)
            s_s[...] = jnp.zeros_like(s_s)
            acc_s[...] = jnp.zeros_like(acc_s)

        w = wbf_s[...]
        hp = hp_s[...]
        asp_row = aspr_s[...]
        adp_col = adp_s[...]

        hs = jnp.dot(fs_ref[...].astype(BF), w,
                     preferred_element_type=jnp.float32)
        as_col = jnp.sum(hs * asrc, axis=1, keepdims=True)
        ad_col = jnp.sum(hs * adst, axis=1, keepdims=True)
        as_row = _row_dot(asrc, hs)

        g1_s[pl.ds(k * BLK, BLK), :] = _sample_side(
            hs, hp, asp_row, jnp.max(asp_row), as_col, ad_col, b).astype(BF)

        ep = _lrelu(adp_col + as_row)
        bmax = _lrelu(jnp.max(as_row) + adp_col)
        new_m = jnp.maximum(m_s[...], bmax)
        scale = jnp.exp(m_s[...] - new_m)
        wp = jnp.exp(ep - new_m)
        s_s[...] = s_s[...] * scale + jnp.sum(wp, axis=1, keepdims=True)
        acc_s[...] = (acc_s[...] * scale
                      + jnp.dot(wp, hs, preferred_element_type=jnp.float32))
        m_s[...] = new_m

        @pl.when(k == K - 1)
        def _():
            e_sp = _lrelu(aspc_s[...] + adp_col)
            fm = jnp.maximum(m_s[...], e_sp)
            sc = jnp.exp(m_s[...] - fm)
            wsp = jnp.exp(e_sp - fm)
            den = s_s[...] * sc + wsp + EPS
            accf = acc_s[...] * sc + wsp * hp
            gp1_s[...] = jnp.maximum(accf / den + b, 0.0)

    @pl.when(l == 1)
    def _layer2():
        asrc = as2_ref[...]
        adst = ad2_ref[...]
        b = b2_ref[...]

        @pl.when(k == 0)
        def _():
            wbf = w2_ref[...].astype(BF)
            wbf_s[...] = wbf
            wfcbf_s[...] = wfc_ref[...].astype(BF)
            hp0 = jnp.dot(gp1_s[...].astype(BF), wbf,
                          preferred_element_type=jnp.float32)
            hp_s[...] = hp0
            aspr_s[...] = _row_dot(asrc, hp0)

        w = wbf_s[...]
        hp = hp_s[...]
        asp_row = aspr_s[...]

        fs = g1_s[pl.ds(k * BLK, BLK), :]
        hs = jnp.dot(fs, w, preferred_element_type=jnp.float32)
        as_col = jnp.sum(hs * asrc, axis=1, keepdims=True)
        ad_col = jnp.sum(hs * adst, axis=1, keepdims=True)

        g = _sample_side(hs, hp, asp_row, jnp.max(asp_row), as_col,
                         ad_col, b)
        h_ref[...] = g
        pred_ref[...] = (jnp.dot(g.astype(BF), wfcbf_s[...],
                                 preferred_element_type=jnp.float32)
                         + bfc_ref[...])


@jax.jit
def _run(x, proxies, W1, a_src1, a_dst1, b1, W2, a_src2, a_dst2, b2,
         W_fc, b_fc):
    C = W_fc.shape[1]

    h2, preds = pl.pallas_call(
        _fused_body,
        grid=(2, K),
        in_specs=[
            pl.BlockSpec((P, D), lambda l, k: (0, 0)),
            pl.BlockSpec((BLK, D), lambda l, k: (k * (1 - l), 0)),
            pl.BlockSpec((D, D), lambda l, k: (0, 0)),
            pl.BlockSpec((1, D), lambda l, k: (0, 0)),
            pl.BlockSpec((1, D), lambda l, k: (0, 0)),
            pl.BlockSpec((1, D), lambda l, k: (0, 0)),
            pl.BlockSpec((D, D), lambda l, k: (0, 0)),
            pl.BlockSpec((1, D), lambda l, k: (0, 0)),
            pl.BlockSpec((1, D), lambda l, k: (0, 0)),
            pl.BlockSpec((1, D), lambda l, k: (0, 0)),
            pl.BlockSpec((D, C), lambda l, k: (0, 0)),
            pl.BlockSpec((1, C), lambda l, k: (0, 0)),
        ],
        out_specs=[
            pl.BlockSpec((BLK, D), lambda l, k: (k * l, 0)),
            pl.BlockSpec((BLK, C), lambda l, k: (k * l, 0)),
        ],
        out_shape=[
            jax.ShapeDtypeStruct((S, D), jnp.float32),
            jax.ShapeDtypeStruct((S, C), jnp.float32),
        ],
        scratch_shapes=[
            pltpu.VMEM((S, D), BF),
            pltpu.VMEM((P, D), jnp.float32),
            pltpu.VMEM((D, D), BF),
            pltpu.VMEM((D, 100), BF),
            pltpu.VMEM((P, D), jnp.float32),
            pltpu.VMEM((1, P), jnp.float32),
            pltpu.VMEM((P, 1), jnp.float32),
            pltpu.VMEM((P, 1), jnp.float32),
            pltpu.VMEM((P, 1), jnp.float32),
            pltpu.VMEM((P, 1), jnp.float32),
            pltpu.VMEM((P, D), jnp.float32),
        ],
    )(proxies, x,
      W1, a_src1[None, :], a_dst1[None, :], b1[None, :],
      W2, a_src2[None, :], a_dst2[None, :], b2[None, :],
      W_fc, b_fc[None, :])

    return preds, h2


def kernel(x, proxies, W1, a_src1, a_dst1, b1, W2, a_src2, a_dst2, b2,
           W_fc, b_fc):
    return _run(x, proxies, W1, a_src1, a_dst1, b1,
                W2, a_src2, a_dst2, b2, W_fc, b_fc)
